# baseline (speedup 1.0000x reference)
import functools

import jax
import jax.numpy as jnp
from jax import lax
from jax.experimental import pallas as pl
from jax.experimental.pallas import tpu as pltpu
from jax.experimental.pallas import tpu_sc as plsc

B = 16384
D = 64

_info = plsc.get_sparse_core_info()
_NC, _NS = _info.num_cores, _info.num_subcores
_NW = _NC * _NS
_BPW = B // _NW
_CHUNK = 128
_NCHUNK = _BPW // _CHUNK


def _gather3_body(cat_i, col_i, fab_i, cat_t, colp, fabp,
                  o_cat, o_col, o_fab,
                  iv0, iv1, iv2, rv_cat, rv_wide, sem_cat, sem_col, sem_fab):
    wid = lax.axis_index("s") * _NC + lax.axis_index("c")
    base = wid * _BPW
    pltpu.sync_copy(cat_i.at[pl.ds(base, _BPW)], iv0)
    pltpu.sync_copy(col_i.at[pl.ds(base, _BPW)], iv1)
    pltpu.sync_copy(fab_i.at[pl.ds(base, _BPW)], iv2)
    cat_copies = []
    col_copies = []
    for j in range(_NCHUNK):
        sl = pl.ds(j * _CHUNK, _CHUNK)
        cat_copies.append(
            pltpu.async_copy(cat_t.at[iv0.at[sl]], rv_cat.at[sl], sem_cat))
        col_copies.append(
            pltpu.async_copy(colp.at[iv1.at[sl]], rv_wide.at[sl], sem_col))
    for c in col_copies:
        c.wait()
    pltpu.sync_copy(rv_wide, o_col.at[pl.ds(base, _BPW)])
    fab_copies = []
    for j in range(_NCHUNK):
        sl = pl.ds(j * _CHUNK, _CHUNK)
        fab_copies.append(
            pltpu.async_copy(fabp.at[iv2.at[sl]], rv_wide.at[sl], sem_fab))
    for c in cat_copies:
        c.wait()
    pltpu.sync_copy(rv_cat, o_cat.at[pl.ds(base, _BPW), pl.ds(0, D)])
    for c in fab_copies:
        c.wait()
    pltpu.sync_copy(rv_wide, o_fab.at[pl.ds(base, _BPW)])


@jax.jit
def _gather3(cat, col, fab, cat_table, colp, fabp):
    mesh = plsc.VectorSubcoreMesh(core_axis_name="c", subcore_axis_name="s")
    f = functools.partial(
        pl.kernel,
        mesh=mesh,
        out_type=[jax.ShapeDtypeStruct((B, 2 * D), jnp.float32)] * 3,
        scratch_types=[pltpu.VMEM((_BPW,), jnp.int32)] * 3
        + [pltpu.VMEM((_BPW, D), jnp.float32),
           pltpu.VMEM((_BPW, 2 * D), jnp.float32)]
        + [pltpu.SemaphoreType.DMA] * 3,
        compiler_params=pltpu.CompilerParams(use_tc_tiling_on_sc=False),
    )(_gather3_body)
    return f(cat, col, fab, cat_table, colp, fabp)


def _fuse_body(x0_ref, x1_ref, x2_ref, wt_ref, b_ref, o_ref):
    wt = wt_ref[...]
    acc = jnp.dot(x0_ref[:, :D], wt[0:D, :], preferred_element_type=jnp.float32)
    acc += jnp.dot(x1_ref[:, :D], wt[D:2 * D, :], preferred_element_type=jnp.float32)
    acc += jnp.dot(x2_ref[:, :D], wt[2 * D:3 * D, :], preferred_element_type=jnp.float32)
    o_ref[...] = acc + b_ref[...]


_BLK = 2048


@jax.jit
def _fuse(x0, x1, x2, wt, b2):
    grid = (B // _BLK,)
    return pl.pallas_call(
        _fuse_body,
        grid=grid,
        in_specs=[
            pl.BlockSpec((_BLK, 2 * D), lambda i: (i, 0)),
            pl.BlockSpec((_BLK, 2 * D), lambda i: (i, 0)),
            pl.BlockSpec((_BLK, 2 * D), lambda i: (i, 0)),
            pl.BlockSpec((3 * D, D), lambda i: (0, 0)),
            pl.BlockSpec((1, D), lambda i: (0, 0)),
        ],
        out_specs=pl.BlockSpec((_BLK, D), lambda i: (i, 0)),
        out_shape=jax.ShapeDtypeStruct((B, D), jnp.float32),
    )(x0, x1, x2, wt, b2)


def kernel(cat, col, fab, cat_table, col_table, fab_table, W, b):
    colp = jnp.pad(col_table, ((0, 0), (0, D)))
    fabp = jnp.pad(fab_table, ((0, 0), (0, D)))
    cat_emb, col_emb, fab_emb = _gather3(
        cat.astype(jnp.int32), col.astype(jnp.int32), fab.astype(jnp.int32),
        cat_table, colp, fabp)
    return _fuse(cat_emb, col_emb, fab_emb, W.T, b.reshape(1, D))


# speedup vs baseline: 1.0199x; 1.0199x over previous
import functools

import jax
import jax.numpy as jnp
from jax import lax
from jax.experimental import pallas as pl
from jax.experimental.pallas import tpu as pltpu
from jax.experimental.pallas import tpu_sc as plsc

B = 16384
D = 64

_info = plsc.get_sparse_core_info()
_NC, _NS = _info.num_cores, _info.num_subcores
_NW = _NC * _NS
_BPW = B // _NW
_CHUNK = 128
_NCHUNK = _BPW // _CHUNK


def _gather3_body(cat_i, col_i, fab_i, cat_t, colp, fabp,
                  o_cat, o_col, o_fab,
                  iv0, iv1, iv2, rv_cat, rv_col, rv_fab,
                  sem_cat, sem_col, sem_fab):
    wid = lax.axis_index("s") * _NC + lax.axis_index("c")
    rbase = wid * _NCHUNK
    base = wid * _BPW
    pltpu.sync_copy(cat_i.at[pl.ds(rbase, _NCHUNK)], iv0)
    pltpu.sync_copy(col_i.at[pl.ds(rbase, _NCHUNK)], iv1)
    pltpu.sync_copy(fab_i.at[pl.ds(rbase, _NCHUNK)], iv2)
    half = _BPW // 2
    nh = _NCHUNK // 2

    def fire_small(tab, iv, rv, sem, pass_i):
        return [
            pltpu.async_copy(tab.at[iv.at[pass_i * nh + j]],
                             rv.at[pl.ds(j * _CHUNK, _CHUNK)], sem)
            for j in range(nh)
        ]

    cat_copies = [
        pltpu.async_copy(cat_t.at[iv0.at[j]],
                         rv_cat.at[pl.ds(j * _CHUNK, _CHUNK)], sem_cat)
        for j in range(_NCHUNK)
    ]
    col_p = fire_small(colp, iv1, rv_col, sem_col, 0)
    fab_p = fire_small(fabp, iv2, rv_fab, sem_fab, 0)
    for c in col_p:
        c.wait()
    pltpu.sync_copy(rv_col, o_col.at[pl.ds(base, half)])
    col_p = fire_small(colp, iv1, rv_col, sem_col, 1)
    for c in fab_p:
        c.wait()
    pltpu.sync_copy(rv_fab, o_fab.at[pl.ds(base, half)])
    fab_p = fire_small(fabp, iv2, rv_fab, sem_fab, 1)
    for c in col_p:
        c.wait()
    pltpu.sync_copy(rv_col, o_col.at[pl.ds(base + half, half)])
    for c in fab_p:
        c.wait()
    pltpu.sync_copy(rv_fab, o_fab.at[pl.ds(base + half, half)])
    for c in cat_copies:
        c.wait()
    pltpu.sync_copy(rv_cat, o_cat.at[pl.ds(base, _BPW), pl.ds(0, D)])


@jax.jit
def _gather3(cat2, col2, fab2, cat_table, colp, fabp):
    mesh = plsc.VectorSubcoreMesh(core_axis_name="c", subcore_axis_name="s")
    f = functools.partial(
        pl.kernel,
        mesh=mesh,
        out_type=[jax.ShapeDtypeStruct((B, 2 * D), jnp.float32)] * 3,
        scratch_types=[pltpu.VMEM((_NCHUNK, _CHUNK), jnp.int32)] * 3
        + [pltpu.VMEM((_BPW, D), jnp.float32),
           pltpu.VMEM((_BPW // 2, 2 * D), jnp.float32),
           pltpu.VMEM((_BPW // 2, 2 * D), jnp.float32)]
        + [pltpu.SemaphoreType.DMA] * 3,
        compiler_params=pltpu.CompilerParams(use_tc_tiling_on_sc=False),
    )(_gather3_body)
    return f(cat2, col2, fab2, cat_table, colp, fabp)


def _fuse_body(x0_ref, x1_ref, x2_ref, wt_ref, b_ref, o_ref):
    wt = wt_ref[...]
    acc = jnp.dot(x0_ref[:, :D], wt[0:D, :], preferred_element_type=jnp.float32)
    acc += jnp.dot(x1_ref[:, :D], wt[D:2 * D, :], preferred_element_type=jnp.float32)
    acc += jnp.dot(x2_ref[:, :D], wt[2 * D:3 * D, :], preferred_element_type=jnp.float32)
    o_ref[...] = acc + b_ref[...]


_BLK = 2048


@jax.jit
def _fuse(x0, x1, x2, wt, b2):
    grid = (B // _BLK,)
    return pl.pallas_call(
        _fuse_body,
        grid=grid,
        in_specs=[
            pl.BlockSpec((_BLK, 2 * D), lambda i: (i, 0)),
            pl.BlockSpec((_BLK, 2 * D), lambda i: (i, 0)),
            pl.BlockSpec((_BLK, 2 * D), lambda i: (i, 0)),
            pl.BlockSpec((3 * D, D), lambda i: (0, 0)),
            pl.BlockSpec((1, D), lambda i: (0, 0)),
        ],
        out_specs=pl.BlockSpec((_BLK, D), lambda i: (i, 0)),
        out_shape=jax.ShapeDtypeStruct((B, D), jnp.float32),
    )(x0, x1, x2, wt, b2)


def kernel(cat, col, fab, cat_table, col_table, fab_table, W, b):
    colp = jnp.pad(col_table, ((0, 0), (0, D)))
    fabp = jnp.pad(fab_table, ((0, 0), (0, D)))
    cat2 = cat.astype(jnp.int32).reshape(B // _CHUNK, _CHUNK)
    col2 = col.astype(jnp.int32).reshape(B // _CHUNK, _CHUNK)
    fab2 = fab.astype(jnp.int32).reshape(B // _CHUNK, _CHUNK)
    cat_emb, col_emb, fab_emb = _gather3(
        cat2, col2, fab2, cat_table, colp, fabp)
    return _fuse(cat_emb, col_emb, fab_emb, W.T, b.reshape(1, D))
